# baseline (speedup 1.0000x reference)
import functools

import jax
import jax.numpy as jnp
from jax import lax
from jax.experimental import pallas as pl
from jax.experimental.pallas import tpu as pltpu
from jax.experimental.pallas import tpu_sc as plsc

D = 64
N = 4096 * 200
NW = 32
PER_W = N // NW
BLK = 128
SUP = 1024
N_SUP = PER_W // SUP
BLKS = SUP // BLK

_mesh = plsc.VectorSubcoreMesh(core_axis_name="c", subcore_axis_name="s")


@functools.partial(
    pl.kernel,
    mesh=_mesh,
    out_type=jax.ShapeDtypeStruct((N, D), jnp.float32),
    compiler_params=pltpu.CompilerParams(use_tc_tiling_on_sc=False),
    scratch_types=[
        pltpu.VMEM((BLKS, BLK), jnp.int32),
        pltpu.VMEM((SUP, D), jnp.float32),
        pltpu.SemaphoreType.DMA,
    ],
)
def _emb_lookup(x_hbm, table_hbm, out_hbm, idx_v, rows_v, gsem):
    wid = lax.axis_index("s") * 2 + lax.axis_index("c")
    base = wid * PER_W

    def body(s, carry):
        off = base + s * SUP
        row = pl.multiple_of(off // BLK, 8)
        pltpu.sync_copy(x_hbm.at[pl.ds(row, BLKS)], idx_v)
        copies = [
            pltpu.async_copy(
                table_hbm.at[idx_v.at[j]],
                rows_v.at[pl.ds(j * BLK, BLK)],
                gsem,
            )
            for j in range(BLKS)
        ]
        for c in copies:
            c.wait()
        pltpu.sync_copy(rows_v, out_hbm.at[pl.ds(off, SUP)])
        return carry

    lax.fori_loop(0, N_SUP, body, 0)


def kernel(x, table):
    xf = x.reshape(N // BLK, BLK)
    out = _emb_lookup(xf, table)
    return out.reshape(x.shape + (D,))
